# baseline (speedup 1.0000x reference)
import functools

import jax
import jax.numpy as jnp
from jax import lax
from jax.experimental import pallas as pl
from jax.experimental.pallas import tpu as pltpu
from jax.experimental.pallas import tpu_sc as plsc

_V = 1_000_000
_B = 16384
_D = 64
_NC = 2
_NS = 16
_NW = _NC * _NS
_NBLK = _V // 128
_BASE_BLKS = _NBLK // _NW
_EXTRA = _NBLK % _NW
_TAIL = _NBLK * 128
_OUT_RING = 32

_mesh = plsc.VectorSubcoreMesh(core_axis_name="c", subcore_axis_name="s")


def _i16():
    return lax.iota(jnp.int32, 16)


def _splat(x):
    return jnp.broadcast_to(x, (16,))


@functools.partial(
    pl.kernel,
    mesh=_mesh,
    out_type=jax.ShapeDtypeStruct((_B, _D), jnp.float32),
    scratch_types=[
        pltpu.VMEM((_B + 16,), jnp.int32),
        pltpu.VMEM((_B + 16,), jnp.int32),
        pltpu.VMEM((_B + 16,), jnp.int32),
        pltpu.VMEM((_D, 128), jnp.float32),
        pltpu.VMEM((_D, 128), jnp.float32),
        pltpu.VMEM((_OUT_RING, _D), jnp.float32),
        pltpu.VMEM((_D, _D), jnp.float32),
        pltpu.SemaphoreType.DMA,
        pltpu.SemaphoreType.DMA,
        pltpu.SemaphoreType.DMA,
    ],
    compiler_params=pltpu.CompilerParams(needs_layout_passes=False),
)
def _gather(idx_hbm, tab_hbm, tail_hbm, out_hbm, idx_v, list_r, list_p,
            ch_a, ch_b, out_st, tail_v, sem_a, sem_b, sem_o):
    wid = lax.axis_index("s") * _NC + lax.axis_index("c")
    s_w = _BASE_BLKS * wid + jnp.minimum(wid, _EXTRA)
    n_w = _BASE_BLKS + (wid < _EXTRA).astype(jnp.int32)
    is_last = (wid == _NW - 1).astype(jnp.int32)

    pltpu.sync_copy(idx_hbm, idx_v.at[pl.ds(0, _B)])
    pltpu.sync_copy(tail_hbm, tail_v)

    def scan_it(i, cnt):
        ivec = idx_v[pl.ds(i * 16, 16)]
        blk = lax.shift_right_logical(ivec, 7)
        m = (blk >= s_w) & (blk < s_w + n_w + is_last)
        plsc.store_compressed(list_r.at[pl.ds(cnt, 16)], ivec, mask=m)
        plsc.store_compressed(list_p.at[pl.ds(cnt, 16)], _i16() + i * 16,
                              mask=m)
        return cnt + plsc.all_reduce_population_count(m)[0]

    cnt = lax.fori_loop(0, _B // 16, scan_it, 0, unroll=False)
    nvec = lax.div(cnt + 15, 16)

    def emit(r, p, m_out, src, l):
        slot = lax.rem(m_out, _OUT_RING)

        @pl.when(m_out >= _OUT_RING)
        def _():
            pltpu.make_async_copy(
                out_st.at[pl.ds(0, 1)], out_hbm.at[pl.ds(0, 1)], sem_o
            ).wait()

        for b in range(_D // 16):
            vals = plsc.load_gather(src, [_i16() + 16 * b, _splat(l)])
            plsc.store_scatter(
                out_st, [_splat(slot), _i16() + 16 * b], vals
            )
        pltpu.async_copy(
            out_st.at[pl.ds(slot, 1)], out_hbm.at[pl.ds(p, 1)], sem_o
        )
        return m_out + 1

    def match_block(blk_id, ch, m_out):
        def jt(j, m_out):
            lr = list_r[pl.ds(j * 16, 16)]
            lp = list_p[pl.ds(j * 16, 16)]
            mv = (lax.shift_right_logical(lr, 7) == blk_id) & (
                _i16() + j * 16 < cnt
            )
            hit = plsc.all_reduce_population_count(mv)[0]

            def do_lanes(m_in):
                mvi = mv.astype(jnp.int32)
                m_cur = m_in
                for u in range(16):
                    r = lr[u]
                    p = lp[u]
                    l = r - blk_id * 128
                    m_cur = lax.cond(
                        mvi[u] != 0,
                        lambda mc: emit(r, p, mc, ch, l),
                        lambda mc: mc,
                        m_cur,
                    )
                return m_cur

            return lax.cond(hit != 0, do_lanes, lambda m: m, m_out)

        return lax.fori_loop(0, nvec, jt, m_out, unroll=False)

    def fire_chunk(c, sb, sem):
        t = pl.multiple_of((s_w + c) * 128, 128)
        pltpu.async_copy(tab_hbm.at[:, pl.ds(t, 128)], sb, sem)

    def wait_chunk(sb, sem):
        pltpu.make_async_copy(
            tab_hbm.at[:, pl.ds(0, 128)], sb, sem
        ).wait()

    fire_chunk(0, ch_a, sem_a)

    def pair(i, m_out):
        ca = 2 * i
        cb = 2 * i + 1

        @pl.when(cb < n_w)
        def _():
            fire_chunk(cb, ch_b, sem_b)

        wait_chunk(ch_a, sem_a)
        m_out = match_block(s_w + ca, ch_a, m_out)

        @pl.when(ca + 2 < n_w)
        def _():
            fire_chunk(ca + 2, ch_a, sem_a)

        def do_b(m):
            wait_chunk(ch_b, sem_b)
            return match_block(s_w + cb, ch_b, m)

        return lax.cond(cb < n_w, do_b, lambda m: m, m_out)

    m_out = lax.fori_loop(0, lax.div(n_w + 1, 2), pair, 0, unroll=False)

    def tail_jt(j, m_out):
        lr = list_r[pl.ds(j * 16, 16)]
        lp = list_p[pl.ds(j * 16, 16)]
        mv = (lr >= _TAIL) & (_i16() + j * 16 < cnt)
        hit = plsc.all_reduce_population_count(mv)[0]

        def do_lanes(m_in):
            mvi = mv.astype(jnp.int32)
            m_cur = m_in
            for u in range(16):
                r = lr[u]
                p = lp[u]
                m_cur = lax.cond(
                    mvi[u] != 0,
                    lambda mc: emit(r, p, mc, tail_v, r - _TAIL),
                    lambda mc: mc,
                    m_cur,
                )
            return m_cur

        return lax.cond(hit != 0, do_lanes, lambda m: m, m_out)

    m_out = lax.fori_loop(0, nvec, tail_jt, m_out, unroll=False)

    def drain(i, _):
        @pl.when(i < jnp.minimum(m_out, _OUT_RING))
        def _():
            pltpu.make_async_copy(
                out_st.at[pl.ds(0, 1)], out_hbm.at[pl.ds(0, 1)], sem_o
            ).wait()
        return ()

    lax.fori_loop(0, _OUT_RING, drain, (), unroll=False)


def kernel(idx, latents):
    idx32 = idx.astype(jnp.int32)
    table_t = latents.T
    tail_t = latents[_TAIL:, :].T
    return _gather(idx32, table_t, tail_t)


# speedup vs baseline: 1.9841x; 1.9841x over previous
import functools

import jax
import jax.numpy as jnp
from jax import lax
from jax.experimental import pallas as pl
from jax.experimental.pallas import tpu as pltpu
from jax.experimental.pallas import tpu_sc as plsc

_V = 1_000_000
_B = 16384
_D = 64
_NC = 2
_NS = 16
_NW = _NC * _NS
_NBLK = _V // 128
_BASE_BLKS = _NBLK // _NW
_EXTRA = _NBLK % _NW
_TAIL = _NBLK * 128
_CB = 4
_CW = _CB * 128
_OUT_RING = 32
_SENTINEL = 300 << 21

_mesh = plsc.VectorSubcoreMesh(core_axis_name="c", subcore_axis_name="s")


def _i16():
    return lax.iota(jnp.int32, 16)


def _splat(x):
    return jnp.broadcast_to(x, (16,))


@functools.partial(
    pl.kernel,
    mesh=_mesh,
    out_type=jax.ShapeDtypeStruct((_B, _D), jnp.float32),
    scratch_types=[
        pltpu.VMEM((_B + 16,), jnp.int32),
        pltpu.VMEM((_B + 32,), jnp.int32),
        pltpu.VMEM((_D, _CW), jnp.float32),
        pltpu.VMEM((_D, _CW), jnp.float32),
        pltpu.VMEM((_OUT_RING, _D), jnp.float32),
        pltpu.VMEM((_D, _D), jnp.float32),
        pltpu.SemaphoreType.DMA,
        pltpu.SemaphoreType.DMA,
        pltpu.SemaphoreType.DMA,
    ],
    compiler_params=pltpu.CompilerParams(needs_layout_passes=False),
)
def _gather(idx_hbm, tab_hbm, tail_hbm, out_hbm, idx_v, list_k,
            ch_a, ch_b, out_st, tail_v, sem_a, sem_b, sem_o):
    wid = lax.axis_index("s") * _NC + lax.axis_index("c")
    s_w = _BASE_BLKS * wid + jnp.minimum(wid, _EXTRA)
    n_w = _BASE_BLKS + (wid < _EXTRA).astype(jnp.int32)
    is_last = (wid == _NW - 1).astype(jnp.int32)

    pltpu.sync_copy(idx_hbm, idx_v.at[pl.ds(0, _B)])
    pltpu.sync_copy(tail_hbm, tail_v)

    def scan_it(i, cnt):
        ivec = idx_v[pl.ds(i * 16, 16)]
        blk = lax.shift_right_logical(ivec, 7)
        m = (blk >= s_w) & (blk < s_w + n_w + is_last)
        key = (
            lax.shift_left(blk - s_w, 21)
            | lax.shift_left(ivec & 127, 14)
            | (_i16() + i * 16)
        )
        plsc.store_compressed(list_k.at[pl.ds(cnt, 16)], key, mask=m)
        return cnt + plsc.all_reduce_population_count(m)[0]

    cnt = lax.fori_loop(0, _B // 16, scan_it, 0, unroll=False)
    list_k[pl.ds(cnt, 16)] = _splat(jnp.int32(_SENTINEL))
    nvec = lax.div(cnt + 15, 16)

    def emit(key, m_out, src, l):
        slot = lax.rem(m_out, _OUT_RING)
        p = key & 16383

        @pl.when(m_out >= _OUT_RING)
        def _():
            pltpu.make_async_copy(
                out_st.at[pl.ds(0, 1)], out_hbm.at[pl.ds(0, 1)], sem_o
            ).wait()

        for b in range(_D // 16):
            vals = plsc.load_gather(src, [_i16() + 16 * b, _splat(l)])
            plsc.store_scatter(
                out_st, [_splat(slot), _i16() + 16 * b], vals
            )
        pltpu.async_copy(
            out_st.at[pl.ds(slot, 1)], out_hbm.at[pl.ds(p, 1)], sem_o
        )
        return m_out + 1

    def match_chunk(ci, ch, m_out):
        def jt(j, m_out):
            kv = list_k[pl.ds(j * 16, 16)]
            mv = lax.shift_right_logical(kv, 23) == ci
            hit = plsc.all_reduce_population_count(mv)[0]

            def do_lanes(m_in):
                mvi = mv.astype(jnp.int32)
                m_cur = m_in
                for u in range(16):
                    key = kv[u]
                    l = (
                        lax.shift_right_logical(key, 21) & (_CB - 1)
                    ) * 128 | (lax.shift_right_logical(key, 14) & 127)
                    m_cur = lax.cond(
                        mvi[u] != 0,
                        lambda mc: emit(key, mc, ch, l),
                        lambda mc: mc,
                        m_cur,
                    )
                return m_cur

            return lax.cond(hit != 0, do_lanes, lambda m: m, m_out)

        return lax.fori_loop(0, nvec, jt, m_out, unroll=False)

    def fire_chunk(ci, sb, sem):
        t = pl.multiple_of((s_w + ci * _CB) * 128, 128)
        pltpu.async_copy(tab_hbm.at[:, pl.ds(t, _CW)], sb, sem)

    def wait_chunk(sb, sem):
        pltpu.make_async_copy(
            tab_hbm.at[:, pl.ds(0, _CW)], sb, sem
        ).wait()

    nch = lax.div(n_w + _CB - 1, _CB)
    fire_chunk(0, ch_a, sem_a)

    def pair(i, m_out):
        ca = 2 * i
        cb = 2 * i + 1

        @pl.when(cb < nch)
        def _():
            fire_chunk(cb, ch_b, sem_b)

        wait_chunk(ch_a, sem_a)
        m_out = match_chunk(ca, ch_a, m_out)

        @pl.when(ca + 2 < nch)
        def _():
            fire_chunk(ca + 2, ch_a, sem_a)

        def do_b(m):
            wait_chunk(ch_b, sem_b)
            return match_chunk(cb, ch_b, m)

        return lax.cond(cb < nch, do_b, lambda m: m, m_out)

    m_out = lax.fori_loop(0, lax.div(nch + 1, 2), pair, 0, unroll=False)

    def tail_jt(j, m_out):
        kv = list_k[pl.ds(j * 16, 16)]
        mv = lax.shift_right_logical(kv, 21) == n_w
        hit = plsc.all_reduce_population_count(mv)[0]

        def do_lanes(m_in):
            mvi = mv.astype(jnp.int32)
            m_cur = m_in
            for u in range(16):
                key = kv[u]
                lt = lax.shift_right_logical(key, 14) & 127
                m_cur = lax.cond(
                    mvi[u] != 0,
                    lambda mc: emit(key, mc, tail_v, lt),
                    lambda mc: mc,
                    m_cur,
                )
            return m_cur

        return lax.cond(hit != 0, do_lanes, lambda m: m, m_out)

    m_out = lax.fori_loop(0, nvec, tail_jt, m_out, unroll=False)

    def drain(i, _):
        @pl.when(i < jnp.minimum(m_out, _OUT_RING))
        def _():
            pltpu.make_async_copy(
                out_st.at[pl.ds(0, 1)], out_hbm.at[pl.ds(0, 1)], sem_o
            ).wait()
        return ()

    lax.fori_loop(0, _OUT_RING, drain, (), unroll=False)


def kernel(idx, latents):
    idx32 = idx.astype(jnp.int32)
    table_t = latents.T
    tail_t = latents[_TAIL:, :].T
    return _gather(idx32, table_t, tail_t)
